# Initial kernel scaffold; baseline (speedup 1.0000x reference)
import jax
import jax.numpy as jnp
from jax.experimental import pallas as pl


def kernel(x, labels, enc_Ws, enc_bs, codebooks, dec_Ws, dec_bs):
    raise NotImplementedError("write your pallas kernel here")


# speedup vs baseline: 1.0543x; 1.0543x over previous
import functools

import jax
import jax.numpy as jnp
from jax.experimental import pallas as pl

_IN_DIM = 768
_LAYERS = [2048, 1024, 512, 256, 128, 64]
_E_DIM = 64
_NUM_LEVELS = 4
_NUM_CODES = 256
_BETA = 0.001
_B = 4096
_TILE = 512

_ENC_DIMS = [_IN_DIM] + _LAYERS + [_E_DIM]
_N_ENC = len(_ENC_DIMS) - 1

_HST = jax.lax.Precision.HIGHEST


def _mlp_kernel(last_prec, x_ref, *refs):
    n = _N_ENC
    ws = refs[0:n]
    bs = refs[n:2 * n]
    out_ref = refs[2 * n]
    h = x_ref[...]
    for i in range(n):
        prec = last_prec if i == n - 1 else None
        h = jnp.dot(h, ws[i][...], precision=prec,
                    preferred_element_type=jnp.float32) + bs[i][...]
        if i != n - 1:
            h = jnp.maximum(h, 0.0)
    out_ref[...] = h


def _run_mlp(x, Ws, bs, d_in, d_out, last_prec=None):
    b2 = [b.reshape(1, -1) for b in bs]
    full = lambda a: pl.BlockSpec(a.shape, lambda i: (0,) * a.ndim)
    return pl.pallas_call(
        functools.partial(_mlp_kernel, last_prec),
        grid=(_B // _TILE,),
        in_specs=[pl.BlockSpec((_TILE, d_in), lambda i: (i, 0))]
        + [full(w) for w in Ws] + [full(b) for b in b2],
        out_specs=[pl.BlockSpec((_TILE, d_out), lambda i: (i, 0))],
        out_shape=[jax.ShapeDtypeStruct((_B, d_out), jnp.float32)],
    )(x, *Ws, *b2)[0]


def _vq_kernel(res_ref, rn_ref, emb_ref, en_ref, idx_ref, q_ref):
    res = res_ref[...]
    emb = emb_ref[...]
    rn = rn_ref[...]
    en = en_ref[...]
    cross = jax.lax.dot_general(res, emb, (((1,), (1,)), ((), ())),
                                preferred_element_type=jnp.float32)
    d = rn + en - 2.0 * cross
    dmin = jnp.min(d, axis=1, keepdims=True)
    iota = jax.lax.broadcasted_iota(jnp.int32, (_TILE, _NUM_CODES), 1)
    idx = jnp.min(jnp.where(d == dmin, iota, _NUM_CODES), axis=1,
                  keepdims=True)
    onehot = (iota == idx).astype(jnp.float32)
    q = jnp.dot(onehot, emb, precision=_HST,
                preferred_element_type=jnp.float32)
    idx_ref[...] = idx
    q_ref[...] = q


def _run_vq(res, rn, emb, en):
    full = lambda a: pl.BlockSpec(a.shape, lambda i: (0,) * a.ndim)
    return pl.pallas_call(
        _vq_kernel,
        grid=(_B // _TILE,),
        in_specs=[pl.BlockSpec((_TILE, _E_DIM), lambda i: (i, 0)),
                  pl.BlockSpec((_TILE, 1), lambda i: (i, 0)),
                  full(emb), full(en)],
        out_specs=[pl.BlockSpec((_TILE, 1), lambda i: (i, 0)),
                   pl.BlockSpec((_TILE, _E_DIM), lambda i: (i, 0))],
        out_shape=[jax.ShapeDtypeStruct((_B, 1), jnp.int32),
                   jax.ShapeDtypeStruct((_B, _E_DIM), jnp.float32)],
    )(res, rn, emb, en)


@jax.jit
def kernel(x, labels, enc_Ws, enc_bs, codebooks, dec_Ws, dec_bs):
    del labels

    x_e = _run_mlp(x, enc_Ws, enc_bs, _IN_DIM, _E_DIM)

    residual = x_e
    x_q = jnp.zeros_like(x_e)
    losses = []
    all_idx = []
    for l in range(_NUM_LEVELS):
        emb = codebooks[l]
        rn = jnp.sum(residual ** 2, axis=1, keepdims=True)
        en = jnp.sum(emb ** 2, axis=1)[None, :]
        idx, q = _run_vq(residual, rn, emb, en)
        commitment_loss = jnp.mean((q - residual) ** 2)
        codebook_loss = jnp.mean((q - residual) ** 2)
        losses.append(codebook_loss + _BETA * commitment_loss)
        q_st = residual + jax.lax.stop_gradient(q - residual)
        residual = residual - q_st
        x_q = x_q + q_st
        all_idx.append(idx[:, 0])
    rq_loss = jnp.mean(jnp.stack(losses))
    indices = jnp.stack(all_idx, axis=-1)

    out = _run_mlp(x_q, dec_Ws, dec_bs, _E_DIM, _IN_DIM)
    return out, rq_loss, indices, x_q
